# Initial kernel scaffold; baseline (speedup 1.0000x reference)
import jax
import jax.numpy as jnp
from jax.experimental import pallas as pl


def kernel(query, reference_points, input_flatten, input_spatial_shapes, W_off, b_off, W_attn, b_attn, W_v, b_v, W_out, b_out):
    raise NotImplementedError("write your pallas kernel here")


# speedup vs baseline: 66.6597x; 66.6597x over previous
import functools

import jax
import jax.numpy as jnp
from jax import lax
from jax.experimental import pallas as pl
from jax.experimental.pallas import tpu as pltpu
from jax.experimental.pallas import tpu_sc as plsc

B = 4
Q = 900
D = 256
NH = 8
NP = 4
HS = 32
WS = 32
DH = D // NH
HW = HS * WS
BQ = B * Q
QP = 912
NBLK = QP // 16
NPAIR = NP * 4


def _prep_body(qT_ref, refT_ref, inT_ref, WcT_ref, bc_ref, WvT_ref, bv_ref,
               idx_ref, cw_ref, vT_ref):
    S = jnp.dot(WcT_ref[...], qT_ref[...],
                preferred_element_type=jnp.float32) + bc_ref[...]
    OX = S[0:32, :]
    OY = S[32:64, :]
    LG = S[64:96, :]

    LGr = LG.reshape(NH, NP, BQ)
    m = jnp.max(LGr, axis=1, keepdims=True)
    e = jnp.exp(LGr - m)
    aw = (e / jnp.sum(e, axis=1, keepdims=True)).reshape(NH * NP, BQ)

    refx = refT_ref[0:1, :]
    refy = refT_ref[1:2, :]
    lx = jnp.clip(refx + OX, 0.0, 1.0) * float(WS - 1)
    ly = jnp.clip(refy + OY, 0.0, 1.0) * float(HS - 1)
    x0f = jnp.floor(lx)
    y0f = jnp.floor(ly)
    x0 = x0f.astype(jnp.int32)
    y0 = y0f.astype(jnp.int32)
    x1 = jnp.minimum(x0 + 1, WS - 1)
    y1 = jnp.minimum(y0 + 1, HS - 1)
    wx1 = lx - x0f
    wx0 = 1.0 - wx1
    wy1 = ly - y0f
    wy0 = 1.0 - wy1

    i00 = y0 * WS + x0
    i01 = y1 * WS + x0
    i10 = y0 * WS + x1
    i11 = y1 * WS + x1
    c00 = wx0 * wy0 * aw
    c01 = wx0 * wy1 * aw
    c10 = wx1 * wy0 * aw
    c11 = wx1 * wy1 * aw

    def corners(a00, a01, a10, a11):
        stk = jnp.concatenate(
            [a.reshape(NH, NP, 1, BQ) for a in (a00, a01, a10, a11)], axis=2)
        return stk.reshape(NH, NPAIR, BQ)

    idx_all = corners(i00, i01, i10, i11)
    cw_all = corners(c00, c01, c10, c11)
    for b in range(B):
        idx_ref[:, :, b, 0:Q] = idx_all[:, :, b * Q:(b + 1) * Q]
        cw_ref[:, :, b, 0:Q] = cw_all[:, :, b * Q:(b + 1) * Q]

    vT_ref[...] = jnp.dot(WvT_ref[...], inT_ref[...],
                          preferred_element_type=jnp.float32) + bv_ref[...]


def _prep(qT, refT, inT, WcT, bc, WvT, bv):
    return pl.pallas_call(
        _prep_body,
        out_shape=(
            jax.ShapeDtypeStruct((NH, NPAIR, B, QP), jnp.int32),
            jax.ShapeDtypeStruct((NH, NPAIR, B, QP), jnp.float32),
            jax.ShapeDtypeStruct((D, B * HW), jnp.float32),
        ),
    )(qT, refT, inT, WcT, bc, WvT, bv)


def _sc_body(vT_hbm, idx_hbm, cw_hbm, out_hbm, table_v, idx_v, cw_v, out_v):
    cid = lax.axis_index("c")
    sid = lax.axis_index("s")
    wid = sid * 2 + cid
    h = wid // B
    b = wid % B

    pltpu.sync_copy(vT_hbm.at[pl.ds(h * DH, DH), pl.ds(b * HW, HW)], table_v)
    pltpu.sync_copy(idx_hbm.at[h, :, b, :], idx_v)
    pltpu.sync_copy(cw_hbm.at[h, :, b, :], cw_v)

    def block(i, carry):
        base = i * 16
        idxs = [jnp.clip(idx_v[j, pl.ds(base, 16)], 0, HW - 1)
                for j in range(NPAIR)]
        ws = [cw_v[j, pl.ds(base, 16)] for j in range(NPAIR)]
        for c in range(DH):
            cv = jnp.full((16,), c, jnp.int32)
            a0 = plsc.load_gather(table_v, [cv, idxs[0]]) * ws[0]
            a1 = plsc.load_gather(table_v, [cv, idxs[1]]) * ws[1]
            for j in range(2, NPAIR, 2):
                a0 = a0 + plsc.load_gather(table_v, [cv, idxs[j]]) * ws[j]
                a1 = a1 + plsc.load_gather(table_v, [cv, idxs[j + 1]]) * ws[j + 1]
            out_v[c, pl.ds(base, 16)] = a0 + a1
        return carry

    lax.fori_loop(0, NBLK, block, 0)
    pltpu.sync_copy(out_v, out_hbm.at[pl.ds(h * DH, DH), b, :])


@functools.cache
def _sc_sample():
    return pl.kernel(
        _sc_body,
        out_type=jax.ShapeDtypeStruct((D, B, QP), jnp.float32),
        mesh=plsc.VectorSubcoreMesh(core_axis_name="c", subcore_axis_name="s",
                                    num_cores=2, num_subcores=16),
        compiler_params=pltpu.CompilerParams(use_tc_tiling_on_sc=False,
                                             needs_layout_passes=False),
        scratch_types=[
            pltpu.VMEM((DH, HW), jnp.float32),
            pltpu.VMEM((NPAIR, QP), jnp.int32),
            pltpu.VMEM((NPAIR, QP), jnp.float32),
            pltpu.VMEM((DH, QP), jnp.float32),
        ],
    )


def _out_body(sT_ref, WoT_ref, bo_ref, o_ref):
    o_ref[...] = jnp.dot(WoT_ref[...], sT_ref[...],
                         preferred_element_type=jnp.float32) + bo_ref[...]


def _outproj(sT, WoT, bo):
    return pl.pallas_call(
        _out_body,
        out_shape=jax.ShapeDtypeStruct((D, B * QP), jnp.float32),
    )(sT, WoT, bo)


def kernel(query, reference_points, input_flatten, input_spatial_shapes,
           W_off, b_off, W_attn, b_attn, W_v, b_v, W_out, b_out):
    qT = query.reshape(BQ, D).T
    refT = reference_points.reshape(BQ, 2).T
    inT = input_flatten.reshape(B * HW, D).T
    WcT = jnp.concatenate(
        [W_off[:, 0::2].T, W_off[:, 1::2].T, W_attn.T], axis=0)
    bc = jnp.concatenate([b_off[0::2], b_off[1::2], b_attn]).reshape(96, 1)
    WvT = W_v.T
    bv = b_v.reshape(D, 1)

    idx, cw, vT = _prep(qT, refT, inT, WcT, bc, WvT, bv)
    sT = _sc_sample()(vT, idx, cw)
    oT = _outproj(sT.reshape(D, B * QP), W_out.T, b_out.reshape(D, 1))
    out = oT.reshape(D, B, QP)[:, :, :Q]
    return jnp.transpose(out, (1, 2, 0))


# speedup vs baseline: 77.2544x; 1.1589x over previous
import functools

import jax
import jax.numpy as jnp
from jax import lax
from jax.experimental import pallas as pl
from jax.experimental.pallas import tpu as pltpu
from jax.experimental.pallas import tpu_sc as plsc

B = 4
Q = 900
D = 256
NH = 8
NP = 4
HS = 32
WS = 32
DH = D // NH
HW = HS * WS
BQ = B * Q
QP = 1024
NBLK = 57
NPAIR = NP * 4
NS = 8


def _prep_body(q_ref, refT_ref, in_ref, WcT_ref, bc_ref, WvT_ref, bv_ref,
               idx_ref, cw_ref, vT_ref):
    S = lax.dot_general(WcT_ref[...], q_ref[...], (((1,), (1,)), ((), ())),
                        preferred_element_type=jnp.float32) + bc_ref[...]
    OX = S[0:32, :]
    OY = S[32:64, :]
    LG = S[64:96, :]

    LGr = LG.reshape(NH, NP, BQ)
    m = jnp.max(LGr, axis=1, keepdims=True)
    e = jnp.exp(LGr - m)
    aw = (e / jnp.sum(e, axis=1, keepdims=True)).reshape(NH * NP, BQ)

    refx = refT_ref[0:1, :]
    refy = refT_ref[1:2, :]
    lx = jnp.clip(refx + OX, 0.0, 1.0) * float(WS - 1)
    ly = jnp.clip(refy + OY, 0.0, 1.0) * float(HS - 1)
    x0f = jnp.floor(lx)
    y0f = jnp.floor(ly)
    x0 = x0f.astype(jnp.int32)
    y0 = y0f.astype(jnp.int32)
    x1 = jnp.minimum(x0 + 1, WS - 1)
    y1 = jnp.minimum(y0 + 1, HS - 1)
    wx1 = lx - x0f
    wx0 = 1.0 - wx1
    wy1 = ly - y0f
    wy0 = 1.0 - wy1

    i00 = y0 * WS + x0
    i01 = y1 * WS + x0
    i10 = y0 * WS + x1
    i11 = y1 * WS + x1
    c00 = wx0 * wy0 * aw
    c01 = wx0 * wy1 * aw
    c10 = wx1 * wy0 * aw
    c11 = wx1 * wy1 * aw

    def corners(a00, a01, a10, a11):
        stk = jnp.concatenate(
            [a.reshape(NH, NP, 1, BQ) for a in (a00, a01, a10, a11)], axis=2)
        return stk.reshape(NH, NPAIR, BQ)

    idx_all = corners(i00, i01, i10, i11)
    cw_all = corners(c00, c01, c10, c11)
    for b in range(B):
        for k in range(NS):
            w = min(Q - k * 128, 128)
            if w <= 0:
                break
            lo = b * Q + k * 128
            idx_ref[:, :, b, k, 0:w] = idx_all[:, :, lo:lo + w]
            cw_ref[:, :, b, k, 0:w] = cw_all[:, :, lo:lo + w]

    vT = lax.dot_general(WvT_ref[...], in_ref[...], (((1,), (1,)), ((), ())),
                         preferred_element_type=jnp.float32) + bv_ref[...]
    for b in range(B):
        for k in range(NS):
            vT_ref[:, b, k, :] = vT[:, b * HW + k * 128:b * HW + (k + 1) * 128]


def _prep(q, refT, in_flat, WcT, bc, WvT, bv):
    return pl.pallas_call(
        _prep_body,
        out_shape=(
            jax.ShapeDtypeStruct((NH, NPAIR, B, NS, 128), jnp.int32),
            jax.ShapeDtypeStruct((NH, NPAIR, B, NS, 128), jnp.float32),
            jax.ShapeDtypeStruct((D, B, NS, 128), jnp.float32),
        ),
    )(q, refT, in_flat, WcT, bc, WvT, bv)


def _sc_body(vT_hbm, idx_hbm, cw_hbm, out_hbm, table_v, idx_v, cw_v, out_v):
    cid = lax.axis_index("c")
    sid = lax.axis_index("s")
    wid = sid * 2 + cid
    h = wid // B
    b = wid % B

    for k in range(NS):
        pltpu.sync_copy(vT_hbm.at[pl.ds(h * DH, DH), b, k, :],
                        table_v.at[:, pl.ds(k * 128, 128)])
    pltpu.sync_copy(idx_hbm.at[h, :, b], idx_v)
    pltpu.sync_copy(cw_hbm.at[h, :, b], cw_v)

    def block(i, carry):
        kk = i // NS
        cc = (i % NS) * 16
        idxs = [jnp.clip(idx_v[j, kk, pl.ds(cc, 16)], 0, HW - 1)
                for j in range(NPAIR)]
        ws = [cw_v[j, kk, pl.ds(cc, 16)] for j in range(NPAIR)]
        for c in range(DH):
            cv = jnp.full((16,), c, jnp.int32)
            a0 = plsc.load_gather(table_v, [cv, idxs[0]]) * ws[0]
            a1 = plsc.load_gather(table_v, [cv, idxs[1]]) * ws[1]
            for j in range(2, NPAIR, 2):
                a0 = a0 + plsc.load_gather(table_v, [cv, idxs[j]]) * ws[j]
                a1 = a1 + plsc.load_gather(table_v, [cv, idxs[j + 1]]) * ws[j + 1]
            out_v[c, kk, pl.ds(cc, 16)] = a0 + a1
        return carry

    lax.fori_loop(0, NBLK, block, 0)
    pltpu.sync_copy(out_v, out_hbm.at[pl.ds(h * DH, DH), b])


@functools.cache
def _sc_sample():
    return pl.kernel(
        _sc_body,
        out_type=jax.ShapeDtypeStruct((D, B, NS, 128), jnp.float32),
        mesh=plsc.VectorSubcoreMesh(core_axis_name="c", subcore_axis_name="s",
                                    num_cores=2, num_subcores=16),
        compiler_params=pltpu.CompilerParams(use_tc_tiling_on_sc=False,
                                             needs_layout_passes=False),
        scratch_types=[
            pltpu.VMEM((DH, HW), jnp.float32),
            pltpu.VMEM((NPAIR, NS, 128), jnp.int32),
            pltpu.VMEM((NPAIR, NS, 128), jnp.float32),
            pltpu.VMEM((DH, NS, 128), jnp.float32),
        ],
    )


def _out_body(s_ref, Wo_ref, bo_ref, o_ref):
    for k in range(NS):
        w = min(Q - k * 128, 128)
        if w <= 0:
            break
        s = s_ref[:, 0, k, :]
        r = lax.dot_general(s, Wo_ref[...], (((0,), (0,)), ((), ())),
                            preferred_element_type=jnp.float32) + bo_ref[...]
        o_ref[0, k * 128:k * 128 + w, :] = r[0:w]


def _outproj(sT, Wo, bo):
    return pl.pallas_call(
        _out_body,
        grid=(B,),
        in_specs=[pl.BlockSpec((D, 1, NS, 128), lambda b: (0, b, 0, 0)),
                  pl.BlockSpec((D, D), lambda b: (0, 0)),
                  pl.BlockSpec((1, D), lambda b: (0, 0))],
        out_specs=pl.BlockSpec((1, Q, D), lambda b: (b, 0, 0)),
        out_shape=jax.ShapeDtypeStruct((B, Q, D), jnp.float32),
    )(sT, Wo, bo)


def kernel(query, reference_points, input_flatten, input_spatial_shapes,
           W_off, b_off, W_attn, b_attn, W_v, b_v, W_out, b_out):
    q = query.reshape(BQ, D)
    refT = reference_points.reshape(BQ, 2).T
    in_flat = input_flatten.reshape(B * HW, D)
    WcT = jnp.concatenate(
        [W_off[:, 0::2].T, W_off[:, 1::2].T, W_attn.T], axis=0)
    bc = jnp.concatenate([b_off[0::2], b_off[1::2], b_attn]).reshape(96, 1)

    idx, cw, vT = _prep(q, refT, in_flat, WcT, bc, W_v.T, b_v.reshape(D, 1))
    sT = _sc_sample()(vT, idx, cw)
    return _outproj(sT, W_out, b_out.reshape(1, D))


# speedup vs baseline: 90.7128x; 1.1742x over previous
import functools

import jax
import jax.numpy as jnp
from jax import lax
from jax.experimental import pallas as pl
from jax.experimental.pallas import tpu as pltpu
from jax.experimental.pallas import tpu_sc as plsc

B = 4
Q = 900
D = 256
NH = 8
NP = 4
HS = 32
WS = 32
DH = D // NH
HW = HS * WS
BQ = B * Q
NBLK = 57
NS = 8


def _prep_body(q_ref, refT_ref, in_ref, WcT_ref, bc_ref, WvT_ref, bv_ref,
               idx_ref, cw_ref, vT_ref):
    S = lax.dot_general(WcT_ref[0], q_ref[0], (((1,), (1,)), ((), ())),
                        preferred_element_type=jnp.float32) + bc_ref[0]
    OX = S[0:32, :]
    OY = S[32:64, :]
    LG = S[64:96, :]

    LGr = LG.reshape(NH, NP, Q)
    m = jnp.max(LGr, axis=1, keepdims=True)
    e = jnp.exp(LGr - m)
    aw = (e / jnp.sum(e, axis=1, keepdims=True)).reshape(NH * NP, Q)

    refx = refT_ref[0, 0:1, :]
    refy = refT_ref[0, 1:2, :]
    lx = jnp.clip(refx + OX, 0.0, 1.0) * float(WS - 1)
    ly = jnp.clip(refy + OY, 0.0, 1.0) * float(HS - 1)
    x0f = jnp.floor(lx)
    y0f = jnp.floor(ly)
    x0 = x0f.astype(jnp.int32)
    y0 = y0f.astype(jnp.int32)
    x1 = jnp.minimum(x0 + 1, WS - 1)
    y1 = jnp.minimum(y0 + 1, HS - 1)
    wx1 = lx - x0f
    wx0 = 1.0 - wx1
    wy1 = ly - y0f
    wy0 = 1.0 - wy1

    idx_c = (y0 * WS + x0, y1 * WS + x0, y0 * WS + x1, y1 * WS + x1)
    cw_c = (wx0 * wy0 * aw, wx0 * wy1 * aw, wx1 * wy0 * aw, wx1 * wy1 * aw)

    for ci in range(4):
        for k in range(NS):
            w = min(Q - k * 128, 128)
            if w > 0:
                idx_ref[ci, :, 0, k, 0:w] = idx_c[ci][:, k * 128:k * 128 + w]
                cw_ref[ci, :, 0, k, 0:w] = cw_c[ci][:, k * 128:k * 128 + w]
        idx_ref[ci, :, 0, NS - 1, Q - (NS - 1) * 128:128] = jnp.zeros(
            (DH, 128 - (Q - (NS - 1) * 128)), jnp.int32)
        cw_ref[ci, :, 0, NS - 1, Q - (NS - 1) * 128:128] = jnp.zeros(
            (DH, 128 - (Q - (NS - 1) * 128)), jnp.float32)

    vT = lax.dot_general(WvT_ref[...], in_ref[0], (((1,), (1,)), ((), ())),
                         preferred_element_type=jnp.float32) + bv_ref[...]
    for k in range(NS):
        vT_ref[:, 0, k, :] = vT[:, k * 128:(k + 1) * 128]


def _prep(q3, refT3, in3, WcT, bc, WvT, bv):
    return pl.pallas_call(
        _prep_body,
        grid=(B,),
        in_specs=[
            pl.BlockSpec((1, Q, D), lambda b: (b, 0, 0)),
            pl.BlockSpec((1, 2, Q), lambda b: (b, 0, 0)),
            pl.BlockSpec((1, HW, D), lambda b: (b, 0, 0)),
            pl.BlockSpec((1, 96, D), lambda b: (0, 0, 0)),
            pl.BlockSpec((1, 96, 1), lambda b: (0, 0, 0)),
            pl.BlockSpec((D, D), lambda b: (0, 0)),
            pl.BlockSpec((D, 1), lambda b: (0, 0)),
        ],
        out_specs=(
            pl.BlockSpec((4, DH, 1, NS, 128), lambda b: (0, 0, b, 0, 0)),
            pl.BlockSpec((4, DH, 1, NS, 128), lambda b: (0, 0, b, 0, 0)),
            pl.BlockSpec((D, 1, NS, 128), lambda b: (0, b, 0, 0)),
        ),
        out_shape=(
            jax.ShapeDtypeStruct((4, DH, B, NS, 128), jnp.int32),
            jax.ShapeDtypeStruct((4, DH, B, NS, 128), jnp.float32),
            jax.ShapeDtypeStruct((D, B, NS, 128), jnp.float32),
        ),
    )(q3, refT3, in3, WcT, bc, WvT, bv)


def _sc_body(vT_hbm, idx_hbm, cw_hbm, out_hbm, table_v, idx_v, cw_v, out_v,
             sem):
    cid = lax.axis_index("c")
    sid = lax.axis_index("s")
    wid = sid * 2 + cid
    h = wid // B
    b = wid % B

    copies = [
        pltpu.async_copy(vT_hbm.at[pl.ds(h * DH, DH), b, k, :],
                         table_v.at[:, pl.ds(k * 128, 128)], sem)
        for k in range(NS)
    ]
    copies.append(pltpu.async_copy(idx_hbm.at[:, pl.ds(h * NP, NP), b],
                                   idx_v, sem))
    copies.append(pltpu.async_copy(cw_hbm.at[:, pl.ds(h * NP, NP), b],
                                   cw_v, sem))
    for cp in copies:
        cp.wait()

    def block(i, carry):
        kk = i // NS
        cc = (i % NS) * 16
        idxs = [idx_v[ci, pi, kk, pl.ds(cc, 16)]
                for ci in range(4) for pi in range(NP)]
        ws = [cw_v[ci, pi, kk, pl.ds(cc, 16)]
              for ci in range(4) for pi in range(NP)]
        for c in range(DH):
            row = table_v.at[c]
            a0 = plsc.load_gather(row, [idxs[0]]) * ws[0]
            a1 = plsc.load_gather(row, [idxs[1]]) * ws[1]
            for j in range(2, NP * 4, 2):
                a0 = a0 + plsc.load_gather(row, [idxs[j]]) * ws[j]
                a1 = a1 + plsc.load_gather(row, [idxs[j + 1]]) * ws[j + 1]
            out_v[c, kk, pl.ds(cc, 16)] = a0 + a1
        return carry

    lax.fori_loop(0, NBLK, block, 0)
    pltpu.sync_copy(out_v, out_hbm.at[pl.ds(h * DH, DH), b])


@functools.cache
def _sc_sample():
    return pl.kernel(
        _sc_body,
        out_type=jax.ShapeDtypeStruct((D, B, NS, 128), jnp.float32),
        mesh=plsc.VectorSubcoreMesh(core_axis_name="c", subcore_axis_name="s",
                                    num_cores=2, num_subcores=16),
        compiler_params=pltpu.CompilerParams(use_tc_tiling_on_sc=False,
                                             needs_layout_passes=False),
        scratch_types=[
            pltpu.VMEM((DH, HW), jnp.float32),
            pltpu.VMEM((4, NP, NS, 128), jnp.int32),
            pltpu.VMEM((4, NP, NS, 128), jnp.float32),
            pltpu.VMEM((DH, NS, 128), jnp.float32),
            pltpu.SemaphoreType.DMA,
        ],
    )


def _out_body(s_ref, Wo_ref, bo_ref, o_ref):
    for k in range(NS):
        w = min(Q - k * 128, 128)
        if w <= 0:
            break
        s = s_ref[:, 0, k, :]
        r = lax.dot_general(s, Wo_ref[...], (((0,), (0,)), ((), ())),
                            preferred_element_type=jnp.float32) + bo_ref[...]
        o_ref[0, k * 128:k * 128 + w, :] = r[0:w]


def _outproj(sT, Wo, bo):
    return pl.pallas_call(
        _out_body,
        grid=(B,),
        in_specs=[pl.BlockSpec((D, 1, NS, 128), lambda b: (0, b, 0, 0)),
                  pl.BlockSpec((D, D), lambda b: (0, 0)),
                  pl.BlockSpec((1, D), lambda b: (0, 0))],
        out_specs=pl.BlockSpec((1, Q, D), lambda b: (b, 0, 0)),
        out_shape=jax.ShapeDtypeStruct((B, Q, D), jnp.float32),
    )(sT, Wo, bo)


def kernel(query, reference_points, input_flatten, input_spatial_shapes,
           W_off, b_off, W_attn, b_attn, W_v, b_v, W_out, b_out):
    refT3 = jnp.transpose(reference_points, (0, 2, 1))
    WcT = jnp.concatenate(
        [W_off[:, 0::2].T, W_off[:, 1::2].T, W_attn.T], axis=0)
    bc = jnp.concatenate([b_off[0::2], b_off[1::2], b_attn]).reshape(1, 96, 1)

    idx, cw, vT = _prep(query, refT3, input_flatten, WcT[None], bc,
                        W_v.T, b_v.reshape(D, 1))
    sT = _sc_sample()(vT, idx, cw)
    return _outproj(sT, W_out, b_out.reshape(1, D))
